# Initial kernel scaffold; baseline (speedup 1.0000x reference)
import jax
import jax.numpy as jnp
from jax.experimental import pallas as pl


def kernel(x, batch, W_ih, W_hh, b_ih, b_hh):
    raise NotImplementedError("write your pallas kernel here")


# speedup vs baseline: 12.2769x; 12.2769x over previous
import functools

import jax
import jax.numpy as jnp
from jax.experimental import pallas as pl
from jax.experimental.pallas import tpu as pltpu

N = 100000
D = 128
B = 512
C = 2000
NB = N // C
STEPS = 3
NEG = -1e30
EPS = 1e-10


def _body(x_ref, b_ref, wih_ref, whh_ref, bih_ref, bhh_ref, out_ref,
          h_ref, c_ref, qst_ref, m_ref, s_ref, v_ref):
    k = pl.program_id(0)
    i = k // (NB + 1)
    j = k % (NB + 1)

    @pl.when(k == 0)
    def _init():
        h_ref[...] = jnp.zeros((B, D), jnp.float32)
        c_ref[...] = jnp.zeros((B, D), jnp.float32)
        qst_ref[...] = jnp.zeros((B, 2 * D), jnp.float32)

    @pl.when((j == 0) & (k > 0))
    def _finalize():
        att = v_ref[...] / (s_ref[...] + EPS)
        qst_ref[:, 0:D] = h_ref[...]
        qst_ref[:, D:2 * D] = att

    @pl.when((j == 0) & (i < STEPS))
    def _lstm():
        gates = (
            jax.lax.dot_general(qst_ref[...], wih_ref[...],
                                (((1,), (1,)), ((), ())),
                                preferred_element_type=jnp.float32)
            + jax.lax.dot_general(h_ref[...], whh_ref[...],
                                  (((1,), (1,)), ((), ())),
                                  preferred_element_type=jnp.float32)
            + bih_ref[...] + bhh_ref[...]
        )
        i_g = jax.nn.sigmoid(gates[:, 0:D])
        f_g = jax.nn.sigmoid(gates[:, D:2 * D])
        g_g = jnp.tanh(gates[:, 2 * D:3 * D])
        o_g = jax.nn.sigmoid(gates[:, 3 * D:4 * D])
        c_new = f_g * c_ref[...] + i_g * g_g
        c_ref[...] = c_new
        h_ref[...] = o_g * jnp.tanh(c_new)
        m_ref[...] = jnp.full((B, D), NEG, jnp.float32)
        s_ref[...] = jnp.zeros((B, D), jnp.float32)
        v_ref[...] = jnp.zeros((B, D), jnp.float32)

    @pl.when((j == 0) & (i == STEPS))
    def _emit():
        out_ref[...] = qst_ref[...]

    @pl.when((j > 0) & (i < STEPS))
    def _chunk():
        xb = x_ref[...]
        bb = b_ref[0]
        seg_ids = jax.lax.broadcasted_iota(jnp.int32, (B, C), 0)
        mt = seg_ids == jnp.broadcast_to(bb, (B, C))
        mt_f = mt.astype(jnp.float32)
        pt = jax.lax.dot_general(h_ref[...], xb, (((1,), (1,)), ((), ())),
                                 preferred_element_type=jnp.float32)
        cmax = jnp.max(jnp.where(mt, pt, NEG), axis=1, keepdims=True)
        m_old = m_ref[...]
        new_m = jnp.maximum(m_old, jnp.broadcast_to(cmax, (B, D)))
        scale = jnp.exp(m_old - new_m)
        nm_col = new_m[:, 0:1]
        p_vec = jnp.sum(mt_f * pt, axis=0, keepdims=True)
        m_gat = jnp.sum(mt_f * jnp.broadcast_to(nm_col, (B, C)),
                        axis=0, keepdims=True)
        e_vec = jnp.exp(p_vec - m_gat)
        w = mt_f * jnp.broadcast_to(e_vec, (B, C))
        s_add = jnp.sum(w, axis=1, keepdims=True)
        s_ref[...] = s_ref[...] * scale + jnp.broadcast_to(s_add, (B, D))
        v_ref[...] = v_ref[...] * scale + jax.lax.dot_general(
            w, xb, (((1,), (0,)), ((), ())),
            preferred_element_type=jnp.float32)
        m_ref[...] = new_m


@jax.jit
def kernel(x, batch, W_ih, W_hh, b_ih, b_hh):
    batch32 = batch.astype(jnp.int32).reshape(NB, 1, C)
    bih = b_ih.reshape(1, 4 * D).astype(jnp.float32)
    bhh = b_hh.reshape(1, 4 * D).astype(jnp.float32)

    def chunk_idx(k):
        return jnp.clip(k % (NB + 1) - 1, 0, NB - 1)

    grid = (STEPS * (NB + 1) + 1,)
    out = pl.pallas_call(
        _body,
        grid=grid,
        in_specs=[
            pl.BlockSpec((C, D), lambda k: (chunk_idx(k), 0)),
            pl.BlockSpec((1, 1, C), lambda k: (chunk_idx(k), 0, 0)),
            pl.BlockSpec((4 * D, 2 * D), lambda k: (0, 0)),
            pl.BlockSpec((4 * D, D), lambda k: (0, 0)),
            pl.BlockSpec((1, 4 * D), lambda k: (0, 0)),
            pl.BlockSpec((1, 4 * D), lambda k: (0, 0)),
        ],
        out_specs=pl.BlockSpec((B, 2 * D), lambda k: (0, 0)),
        out_shape=jax.ShapeDtypeStruct((B, 2 * D), jnp.float32),
        scratch_shapes=[
            pltpu.VMEM((B, D), jnp.float32),
            pltpu.VMEM((B, D), jnp.float32),
            pltpu.VMEM((B, 2 * D), jnp.float32),
            pltpu.VMEM((B, D), jnp.float32),
            pltpu.VMEM((B, D), jnp.float32),
            pltpu.VMEM((B, D), jnp.float32),
        ],
    )(x, batch32, W_ih, W_hh, bih, bhh)
    return out
